# baseline (speedup 1.0000x reference)
import jax
import jax.numpy as jnp
from jax.experimental import pallas as pl
from jax.experimental.pallas import tpu as pltpu

D_MODEL = 1024
CAP = 64
BATCH = 4096
BMA = 512
NBA = BATCH // BMA
BM = 256
NB = BATCH // BM

NEG = float(jnp.finfo(jnp.float32).min)


def _proj_kernel(x_ref, wup_ref, bup_ref, wimp_ref, bimp_ref, u_ref, p_ref):
    x = x_ref[...]
    u = jax.lax.dot_general(x, wup_ref[...], (((1,), (1,)), ((), ())),
                            preferred_element_type=jnp.float32,
                            precision=jax.lax.Precision.HIGHEST)
    u_ref[...] = u + bup_ref[...]
    z = jax.lax.dot_general(wimp_ref[...], x, (((1,), (1,)), ((), ())),
                            preferred_element_type=jnp.float32,
                            precision=jax.lax.Precision.HIGHEST)
    p = jax.nn.sigmoid(z + bimp_ref[0, 0])
    p_ref[...] = p.reshape(1, 1, BMA)


def _mem_kernel(p_ref, u_ref, out_ref, slots_ref, state_ref, imp_rows, lw_rows):
    b = pl.program_id(0)
    t0 = b * BM

    @pl.when(b == 0)
    def _init():
        slots_ref[...] = jnp.zeros((CAP, D_MODEL), jnp.float32)
        state_ref[0:1, :] = jnp.zeros((1, CAP), jnp.float32)
        state_ref[1:2, :] = jnp.zeros((1, CAP), jnp.float32)
        state_ref[2:3, :] = jnp.full((1, CAP), -1.0, jnp.float32)

    lane = jax.lax.broadcasted_iota(jnp.int32, (1, CAP), 1)
    lane_f = lane.astype(jnp.float32)

    def body(i, carry):
        imp, ages, lw = carry
        t = t0 + i
        t_f = t.astype(jnp.float32)
        p = p_ref[b, 0, i]
        m = jnp.min(imp, axis=1, keepdims=True)
        cand = jnp.where(imp == m, lane, CAP)
        amin = jnp.min(cand, axis=1, keepdims=True)
        fill = t < CAP
        idx = jnp.where(fill, t, amin)
        should = jnp.logical_or(fill, p > 0.1)
        sel = jnp.logical_and(lane == idx, should)
        imp = jnp.where(sel, jnp.maximum(0.1, p), imp)
        ages = jnp.where(sel, 0.0, ages) + 1.0
        imp = jnp.where(ages > 50.0, imp * 0.9, imp)
        lw = jnp.where(sel, t_f, lw)
        imp_rows[pl.ds(i, 1), :] = imp
        lw_rows[pl.ds(i, 1), :] = lw
        return imp, ages, lw

    imp0 = state_ref[0:1, :]
    ages0 = state_ref[1:2, :]
    lw0 = state_ref[2:3, :]
    impf, agesf, lwf = jax.lax.fori_loop(0, BM, body, (imp0, ages0, lw0))
    state_ref[0:1, :] = impf
    state_ref[1:2, :] = agesf
    state_ref[2:3, :] = lwf

    IMP = imp_rows[...]
    LW = lw_rows[...]
    occ = LW >= 0.0
    logits = jnp.where(occ, IMP, NEG)
    mx = jnp.max(logits, axis=1, keepdims=True)
    e = jnp.exp(logits - mx)
    w = e / jnp.sum(e, axis=1, keepdims=True)

    LWi = LW.astype(jnp.int32)
    in_blk = LWi >= t0
    M1 = jnp.where(in_blk, 0.0, w)
    srow = jax.lax.broadcasted_iota(jnp.int32, (BM, CAP), 0) + t0
    H_T = (LWi == srow).astype(jnp.float32)
    WG = jax.lax.dot_general(w, H_T, (((1,), (1,)), ((), ())),
                             preferred_element_type=jnp.float32,
                             precision=jax.lax.Precision.HIGHEST)
    LG = jax.lax.dot_general(LW, H_T, (((1,), (1,)), ((), ())),
                             preferred_element_type=jnp.float32,
                             precision=jax.lax.Precision.HIGHEST)
    krow = (jax.lax.broadcasted_iota(jnp.int32, (BM, BM), 1)
            + t0).astype(jnp.float32)
    M2 = jnp.where(jnp.abs(LG - krow) < 0.5, WG, 0.0)
    U = u_ref[...]
    out = jax.lax.dot_general(M1, slots_ref[...], (((1,), (0,)), ((), ())),
                              preferred_element_type=jnp.float32,
                            precision=jax.lax.Precision.HIGHEST)
    out = out + jax.lax.dot_general(M2, U, (((1,), (0,)), ((), ())),
                                    preferred_element_type=jnp.float32,
                            precision=jax.lax.Precision.HIGHEST)
    out_ref[...] = out

    lwi = lwf.astype(jnp.int32)
    ST = (lwi == (jax.lax.broadcasted_iota(jnp.int32, (BM, CAP), 0)
                  + t0)).astype(jnp.float32)
    G = jax.lax.dot_general(ST, U, (((0,), (0,)), ((), ())),
                            preferred_element_type=jnp.float32,
                            precision=jax.lax.Precision.HIGHEST)
    ir = jax.lax.broadcasted_iota(jnp.int32, (CAP, CAP), 0)
    ic = jax.lax.broadcasted_iota(jnp.int32, (CAP, CAP), 1)
    keep = jnp.logical_and(ir == ic, lwi < t0).astype(jnp.float32)
    slots_ref[...] = G + jax.lax.dot_general(
        keep, slots_ref[...], (((1,), (0,)), ((), ())),
        preferred_element_type=jnp.float32,
                            precision=jax.lax.Precision.HIGHEST)


def _stage_a(x, W_up, b_up, W_imp, b_imp, interpret=False):
    return pl.pallas_call(
        _proj_kernel,
        grid=(NBA,),
        in_specs=[
            pl.BlockSpec((BMA, D_MODEL), lambda b: (b, 0)),
            pl.BlockSpec((D_MODEL, D_MODEL), lambda b: (0, 0)),
            pl.BlockSpec((1, D_MODEL), lambda b: (0, 0)),
            pl.BlockSpec((1, D_MODEL), lambda b: (0, 0)),
            pl.BlockSpec(memory_space=pltpu.SMEM),
        ],
        out_specs=[
            pl.BlockSpec((BMA, D_MODEL), lambda b: (b, 0)),
            pl.BlockSpec((1, 1, BMA), lambda b: (b, 0, 0)),
        ],
        out_shape=[
            jax.ShapeDtypeStruct((BATCH, D_MODEL), jnp.float32),
            jax.ShapeDtypeStruct((NBA, 1, BMA), jnp.float32),
        ],
        interpret=interpret,
    )(x, W_up, b_up.reshape(1, D_MODEL), W_imp, b_imp.reshape(1, 1))


def _stage_b(P, U, interpret=False):
    return pl.pallas_call(
        _mem_kernel,
        grid=(NB,),
        in_specs=[
            pl.BlockSpec(memory_space=pltpu.SMEM),
            pl.BlockSpec((BM, D_MODEL), lambda b: (b, 0)),
        ],
        out_specs=pl.BlockSpec((BM, D_MODEL), lambda b: (b, 0)),
        out_shape=jax.ShapeDtypeStruct((BATCH, D_MODEL), jnp.float32),
        scratch_shapes=[
            pltpu.VMEM((CAP, D_MODEL), jnp.float32),
            pltpu.VMEM((8, CAP), jnp.float32),
            pltpu.VMEM((BM, CAP), jnp.float32),
            pltpu.VMEM((BM, CAP), jnp.float32),
        ],
        interpret=interpret,
    )(P, U)


def kernel(x, W_up, b_up, W_imp, b_imp):
    U, P = _stage_a(x, W_up, b_up, W_imp, b_imp)
    return _stage_b(P.reshape(NB, 1, BM), U)


# speedup vs baseline: 3.9915x; 3.9915x over previous
import functools

import jax
import jax.numpy as jnp
from jax import lax
from jax.experimental import pallas as pl
from jax.experimental.pallas import tpu as pltpu
from jax.experimental.pallas import tpu_sc as plsc

D_MODEL = 1024
CAP = 64
BATCH = 4096
BMA = 512
NBA = BATCH // BMA
BM = 256
NB = BATCH // BM
SB = 512
NSB = BATCH // SB
L = 16
NV = CAP // L

NEG = float(jnp.finfo(jnp.float32).min)
HIGHEST = jax.lax.Precision.HIGHEST


def _proj_kernel(x_ref, wup_ref, bup_ref, wimp_ref, bimp_ref, u_ref, p_ref):
    x = x_ref[...]
    u = jax.lax.dot_general(x, wup_ref[...], (((1,), (1,)), ((), ())),
                            preferred_element_type=jnp.float32,
                            precision=HIGHEST)
    u_ref[...] = u + bup_ref[...]
    z = jax.lax.dot_general(wimp_ref[...], x, (((1,), (1,)), ((), ())),
                            preferred_element_type=jnp.float32,
                            precision=HIGHEST)
    p = jax.nn.sigmoid(z + bimp_ref[0, 0])
    p_ref[...] = p.reshape(1, 1, BMA)


def _stage_a(x, W_up, b_up, W_imp, b_imp, interpret=False):
    return pl.pallas_call(
        _proj_kernel,
        grid=(NBA,),
        in_specs=[
            pl.BlockSpec((BMA, D_MODEL), lambda b: (b, 0)),
            pl.BlockSpec((D_MODEL, D_MODEL), lambda b: (0, 0)),
            pl.BlockSpec((1, D_MODEL), lambda b: (0, 0)),
            pl.BlockSpec((1, D_MODEL), lambda b: (0, 0)),
            pl.BlockSpec(memory_space=pltpu.SMEM),
        ],
        out_specs=[
            pl.BlockSpec((BMA, D_MODEL), lambda b: (b, 0)),
            pl.BlockSpec((1, 1, BMA), lambda b: (b, 0, 0)),
        ],
        out_shape=[
            jax.ShapeDtypeStruct((BATCH, D_MODEL), jnp.float32),
            jax.ShapeDtypeStruct((NBA, 1, BMA), jnp.float32),
        ],
        interpret=interpret,
    )(x, W_up, b_up.reshape(1, D_MODEL), W_imp, b_imp.reshape(1, 1))


def _sc_recurrence(P):
    mesh = plsc.VectorSubcoreMesh(core_axis_name="c", subcore_axis_name="s")

    @functools.partial(
        pl.kernel,
        mesh=mesh,
        out_type=[
            jax.ShapeDtypeStruct((BATCH * CAP,), jnp.float32),
            jax.ShapeDtypeStruct((BATCH * CAP,), jnp.float32),
        ],
        scratch_types=[
            pltpu.VMEM((BATCH,), jnp.float32),
            pltpu.VMEM((SB * CAP,), jnp.float32),
            pltpu.VMEM((SB * CAP,), jnp.float32),
        ],
    )
    def run(p_hbm, imp_hbm, lw_hbm, p_v, impbuf, lwbuf):
        cid = lax.axis_index("c")
        sid = lax.axis_index("s")

        def gath(x, idx):
            return lax.gather(
                x, idx.reshape(L, 1),
                lax.GatherDimensionNumbers(
                    offset_dims=(), collapsed_slice_dims=(0,),
                    start_index_map=(0,)),
                slice_sizes=(1,),
                mode=lax.GatherScatterMode.PROMISE_IN_BOUNDS)

        def lane_min_splat(x):
            iota = lax.broadcasted_iota(jnp.int32, (L,), 0)
            for k in range(4):
                x = jnp.minimum(x, gath(x, iota ^ (1 << k)))
            return x

        @pl.when(jnp.logical_and(cid == 0, sid == 0))
        def _():
            pltpu.sync_copy(p_hbm, p_v)
            lanes = [
                lax.broadcasted_iota(jnp.int32, (L,), 0) + L * k
                for k in range(NV)
            ]
            zero = jnp.zeros((L,), jnp.float32)

            def step_in_chunk(bk, i, carry):
                imp = carry[0:NV]
                ages = carry[NV:2 * NV]
                lw = carry[2 * NV:3 * NV]
                t = bk * SB + i
                t_f = t.astype(jnp.float32)
                cb = (t // L) * L
                chunk = p_v[pl.ds(cb, L)]
                p_s = lax.gather(
                    chunk,
                    jnp.full((L, 1), t - cb, jnp.int32),
                    lax.GatherDimensionNumbers(
                        offset_dims=(),
                        collapsed_slice_dims=(0,),
                        start_index_map=(0,)),
                    slice_sizes=(1,),
                    mode=lax.GatherScatterMode.PROMISE_IN_BOUNDS)
                m = jnp.minimum(jnp.minimum(imp[0], imp[1]),
                                jnp.minimum(imp[2], imp[3]))
                m_s = lane_min_splat(m)
                cand = [jnp.where(imp[k] == m_s, lanes[k], CAP)
                        for k in range(NV)]
                c = jnp.minimum(jnp.minimum(cand[0], cand[1]),
                                jnp.minimum(cand[2], cand[3]))
                idx_s = lane_min_splat(c)
                one_i = jnp.ones((L,), jnp.int32)
                zero_i = jnp.zeros((L,), jnp.int32)
                should_i = jnp.maximum(
                    jnp.where(p_s > 0.1, one_i, zero_i),
                    jnp.where(m_s <= 0.0, one_i, zero_i))
                newimp = jnp.maximum(p_s, 0.1)
                out = []
                base = i * CAP
                for k in range(NV):
                    sel_i = jnp.where(lanes[k] == idx_s,
                                      one_i, zero_i) * should_i
                    ik = jnp.where(sel_i == 1, newimp, imp[k])
                    ak = jnp.where(sel_i >= 1, zero, ages[k]) + 1.0
                    ik = jnp.where(ak > 50.0, ik * 0.9, ik)
                    wk = jnp.where(sel_i > 0, jnp.full((L,), t_f), lw[k])
                    impbuf[pl.ds(base + L * k, L)] = ik
                    lwbuf[pl.ds(base + L * k, L)] = wk
                    out.append((ik, ak, wk))
                return tuple([o[0] for o in out] + [o[1] for o in out]
                             + [o[2] for o in out])

            carry = tuple([jnp.zeros((L,), jnp.float32)] * (2 * NV)
                          + [jnp.full((L,), -1.0)] * NV)
            for bk in range(NSB):
                carry = lax.fori_loop(
                    0, SB, functools.partial(step_in_chunk, bk), carry)
                pltpu.sync_copy(
                    impbuf, imp_hbm.at[pl.ds(bk * SB * CAP, SB * CAP)])
                pltpu.sync_copy(
                    lwbuf, lw_hbm.at[pl.ds(bk * SB * CAP, SB * CAP)])

    impf, lwf = run(P)
    return impf.reshape(BATCH, CAP), lwf.reshape(BATCH, CAP)


def _combine_kernel(imp_ref, lw_ref, u_ref, out_ref, slots_ref):
    b = pl.program_id(0)
    t0 = b * BM

    @pl.when(b == 0)
    def _init():
        slots_ref[...] = jnp.zeros((CAP, D_MODEL), jnp.float32)

    IMP = imp_ref[...]
    LW = lw_ref[...]
    occ = LW >= 0.0
    logits = jnp.where(occ, IMP, NEG)
    mx = jnp.max(logits, axis=1, keepdims=True)
    e = jnp.exp(logits - mx)
    w = e / jnp.sum(e, axis=1, keepdims=True)

    LWi = LW.astype(jnp.int32)
    in_blk = LWi >= t0
    M1 = jnp.where(in_blk, 0.0, w)
    srow = jax.lax.broadcasted_iota(jnp.int32, (BM, CAP), 0) + t0
    H_T = (LWi == srow).astype(jnp.float32)
    WG = jax.lax.dot_general(w, H_T, (((1,), (1,)), ((), ())),
                             preferred_element_type=jnp.float32,
                             precision=HIGHEST)
    LG = jax.lax.dot_general(LW, H_T, (((1,), (1,)), ((), ())),
                             preferred_element_type=jnp.float32,
                             precision=HIGHEST)
    krow = (jax.lax.broadcasted_iota(jnp.int32, (BM, BM), 1)
            + t0).astype(jnp.float32)
    M2 = jnp.where(jnp.abs(LG - krow) < 0.5, WG, 0.0)
    U = u_ref[...]
    out = jax.lax.dot_general(M1, slots_ref[...], (((1,), (0,)), ((), ())),
                              preferred_element_type=jnp.float32,
                              precision=HIGHEST)
    out = out + jax.lax.dot_general(M2, U, (((1,), (0,)), ((), ())),
                                    preferred_element_type=jnp.float32,
                                    precision=HIGHEST)
    out_ref[...] = out

    lwi_last = LWi[BM - 1:BM, :]
    ST = (lwi_last == (jax.lax.broadcasted_iota(jnp.int32, (BM, CAP), 0)
                       + t0)).astype(jnp.float32)
    G = jax.lax.dot_general(ST, U, (((0,), (0,)), ((), ())),
                            preferred_element_type=jnp.float32,
                            precision=HIGHEST)
    ir = jax.lax.broadcasted_iota(jnp.int32, (CAP, CAP), 0)
    ic = jax.lax.broadcasted_iota(jnp.int32, (CAP, CAP), 1)
    keep = jnp.logical_and(ir == ic, lwi_last < t0).astype(jnp.float32)
    slots_ref[...] = G + jax.lax.dot_general(
        keep, slots_ref[...], (((1,), (0,)), ((), ())),
        preferred_element_type=jnp.float32, precision=HIGHEST)


def _stage_c(IMP, LW, U, interpret=False):
    return pl.pallas_call(
        _combine_kernel,
        grid=(NB,),
        in_specs=[
            pl.BlockSpec((BM, CAP), lambda b: (b, 0)),
            pl.BlockSpec((BM, CAP), lambda b: (b, 0)),
            pl.BlockSpec((BM, D_MODEL), lambda b: (b, 0)),
        ],
        out_specs=pl.BlockSpec((BM, D_MODEL), lambda b: (b, 0)),
        out_shape=jax.ShapeDtypeStruct((BATCH, D_MODEL), jnp.float32),
        scratch_shapes=[
            pltpu.VMEM((CAP, D_MODEL), jnp.float32),
        ],
        interpret=interpret,
    )(IMP, LW, U)


def kernel(x, W_up, b_up, W_imp, b_imp):
    U, P = _stage_a(x, W_up, b_up, W_imp, b_imp)
    IMP, LW = _sc_recurrence(P.reshape(BATCH))
    return _stage_c(IMP, LW, U)


# speedup vs baseline: 4.9419x; 1.2381x over previous
import functools

import jax
import jax.numpy as jnp
from jax import lax
from jax.experimental import pallas as pl
from jax.experimental.pallas import tpu as pltpu
from jax.experimental.pallas import tpu_sc as plsc

D_MODEL = 1024
CAP = 64
BATCH = 4096
BMA = 512
NBA = BATCH // BMA
BM = 256
NB = BATCH // BM
SB = 512
NSB = BATCH // SB
L = 16
NV = CAP // L

NEG = float(jnp.finfo(jnp.float32).min)
HIGHEST = jax.lax.Precision.HIGHEST


def _imp_kernel(x_ref, wimp_ref, bimp_ref, p_ref):
    z = jax.lax.dot_general(wimp_ref[...], x_ref[...], (((1,), (1,)), ((), ())),
                            preferred_element_type=jnp.float32,
                            precision=HIGHEST)
    p = jax.nn.sigmoid(z + bimp_ref[0, 0])
    p_ref[...] = p.reshape(1, 1, BMA)


def _stage_p(x, W_imp, b_imp, interpret=False):
    return pl.pallas_call(
        _imp_kernel,
        grid=(NBA,),
        in_specs=[
            pl.BlockSpec((BMA, D_MODEL), lambda b: (b, 0)),
            pl.BlockSpec((1, D_MODEL), lambda b: (0, 0)),
            pl.BlockSpec(memory_space=pltpu.SMEM),
        ],
        out_specs=pl.BlockSpec((1, 1, BMA), lambda b: (b, 0, 0)),
        out_shape=jax.ShapeDtypeStruct((NBA, 1, BMA), jnp.float32),
        interpret=interpret,
    )(x, W_imp, b_imp.reshape(1, 1))


def _up_kernel(x_ref, wup_ref, bup_ref, u_ref):
    u = jax.lax.dot_general(x_ref[...], wup_ref[...], (((1,), (1,)), ((), ())),
                            preferred_element_type=jnp.float32,
                            precision=HIGHEST)
    u_ref[...] = u + bup_ref[...]


def _stage_u(x, W_up, b_up, interpret=False):
    return pl.pallas_call(
        _up_kernel,
        grid=(NBA,),
        in_specs=[
            pl.BlockSpec((BMA, D_MODEL), lambda b: (b, 0)),
            pl.BlockSpec((D_MODEL, D_MODEL), lambda b: (0, 0)),
            pl.BlockSpec((1, D_MODEL), lambda b: (0, 0)),
        ],
        out_specs=pl.BlockSpec((BMA, D_MODEL), lambda b: (b, 0)),
        out_shape=jax.ShapeDtypeStruct((BATCH, D_MODEL), jnp.float32),
        interpret=interpret,
    )(x, W_up, b_up.reshape(1, D_MODEL))


def _sc_recurrence(P):
    mesh = plsc.VectorSubcoreMesh(core_axis_name="c", subcore_axis_name="s")

    @functools.partial(
        pl.kernel,
        mesh=mesh,
        out_type=[
            jax.ShapeDtypeStruct((BATCH * CAP,), jnp.float32),
            jax.ShapeDtypeStruct((BATCH * CAP,), jnp.float32),
        ],
        scratch_types=[
            pltpu.VMEM((BATCH,), jnp.float32),
            pltpu.VMEM((SB * CAP,), jnp.float32),
            pltpu.VMEM((SB * CAP,), jnp.float32),
        ],
    )
    def run(p_hbm, imp_hbm, lw_hbm, p_v, impbuf, lwbuf):
        cid = lax.axis_index("c")
        sid = lax.axis_index("s")

        def gath(x, idx):
            return lax.gather(
                x, idx.reshape(L, 1),
                lax.GatherDimensionNumbers(
                    offset_dims=(), collapsed_slice_dims=(0,),
                    start_index_map=(0,)),
                slice_sizes=(1,),
                mode=lax.GatherScatterMode.PROMISE_IN_BOUNDS)

        def lane_min_splat(x):
            iota = lax.broadcasted_iota(jnp.int32, (L,), 0)
            for k in range(4):
                x = jnp.minimum(x, gath(x, iota ^ (1 << k)))
            return x

        @pl.when(jnp.logical_and(cid == 0, sid == 0))
        def _():
            pltpu.sync_copy(p_hbm, p_v)
            lanes = [
                lax.broadcasted_iota(jnp.int32, (L,), 0) + L * k
                for k in range(NV)
            ]
            zero = jnp.zeros((L,), jnp.float32)

            def step_in_chunk(bk, i, carry):
                imp = carry[0:NV]
                ages = carry[NV:2 * NV]
                lw = carry[2 * NV:3 * NV]
                t = bk * SB + i
                t_f = t.astype(jnp.float32)
                cb = (t // L) * L
                chunk = p_v[pl.ds(cb, L)]
                p_s = lax.gather(
                    chunk,
                    jnp.full((L, 1), t - cb, jnp.int32),
                    lax.GatherDimensionNumbers(
                        offset_dims=(),
                        collapsed_slice_dims=(0,),
                        start_index_map=(0,)),
                    slice_sizes=(1,),
                    mode=lax.GatherScatterMode.PROMISE_IN_BOUNDS)
                m = jnp.minimum(jnp.minimum(imp[0], imp[1]),
                                jnp.minimum(imp[2], imp[3]))
                m_s = lane_min_splat(m)
                cand = [jnp.where(imp[k] == m_s, lanes[k], CAP)
                        for k in range(NV)]
                c = jnp.minimum(jnp.minimum(cand[0], cand[1]),
                                jnp.minimum(cand[2], cand[3]))
                idx_s = lane_min_splat(c)
                one_i = jnp.ones((L,), jnp.int32)
                zero_i = jnp.zeros((L,), jnp.int32)
                should_i = jnp.maximum(
                    jnp.where(p_s > 0.1, one_i, zero_i),
                    jnp.where(m_s <= 0.0, one_i, zero_i))
                newimp = jnp.maximum(p_s, 0.1)
                out = []
                base = i * CAP
                for k in range(NV):
                    sel_i = jnp.where(lanes[k] == idx_s,
                                      one_i, zero_i) * should_i
                    ik = jnp.where(sel_i == 1, newimp, imp[k])
                    ak = jnp.where(sel_i >= 1, zero, ages[k]) + 1.0
                    ik = jnp.where(ak > 50.0, ik * 0.9, ik)
                    wk = jnp.where(sel_i > 0, jnp.full((L,), t_f), lw[k])
                    impbuf[pl.ds(base + L * k, L)] = ik
                    lwbuf[pl.ds(base + L * k, L)] = wk
                    out.append((ik, ak, wk))
                return tuple([o[0] for o in out] + [o[1] for o in out]
                             + [o[2] for o in out])

            carry = tuple([jnp.zeros((L,), jnp.float32)] * (2 * NV)
                          + [jnp.full((L,), -1.0)] * NV)
            for bk in range(NSB):
                carry = lax.fori_loop(
                    0, SB, functools.partial(step_in_chunk, bk), carry)
                pltpu.sync_copy(
                    impbuf, imp_hbm.at[pl.ds(bk * SB * CAP, SB * CAP)])
                pltpu.sync_copy(
                    lwbuf, lw_hbm.at[pl.ds(bk * SB * CAP, SB * CAP)])

    impf, lwf = run(P)
    return impf.reshape(BATCH, CAP), lwf.reshape(BATCH, CAP)


def _combine_kernel(imp_ref, lw_ref, u_ref, out_ref, slots_ref):
    b = pl.program_id(0)
    t0 = b * BM

    @pl.when(b == 0)
    def _init():
        slots_ref[...] = jnp.zeros((CAP, D_MODEL), jnp.float32)

    IMP = imp_ref[...]
    LW = lw_ref[...]
    occ = LW >= 0.0
    logits = jnp.where(occ, IMP, NEG)
    mx = jnp.max(logits, axis=1, keepdims=True)
    e = jnp.exp(logits - mx)
    w = e / jnp.sum(e, axis=1, keepdims=True)

    LWi = LW.astype(jnp.int32)
    in_blk = LWi >= t0
    M1 = jnp.where(in_blk, 0.0, w)
    srow = jax.lax.broadcasted_iota(jnp.int32, (BM, CAP), 0) + t0
    H_T = (LWi == srow).astype(jnp.float32)
    WG = jax.lax.dot_general(w, H_T, (((1,), (1,)), ((), ())),
                             preferred_element_type=jnp.float32,
                             precision=HIGHEST)
    LG = jax.lax.dot_general(LW, H_T, (((1,), (1,)), ((), ())),
                             preferred_element_type=jnp.float32,
                             precision=HIGHEST)
    krow = (jax.lax.broadcasted_iota(jnp.int32, (BM, BM), 1)
            + t0).astype(jnp.float32)
    M2 = jnp.where(jnp.abs(LG - krow) < 0.5, WG, 0.0)
    U = u_ref[...]
    out = jax.lax.dot_general(M1, slots_ref[...], (((1,), (0,)), ((), ())),
                              preferred_element_type=jnp.float32,
                              precision=HIGHEST)
    out = out + jax.lax.dot_general(M2, U, (((1,), (0,)), ((), ())),
                                    preferred_element_type=jnp.float32,
                                    precision=HIGHEST)
    out_ref[...] = out

    lwi_last = LWi[BM - 1:BM, :]
    ST = (lwi_last == (jax.lax.broadcasted_iota(jnp.int32, (BM, CAP), 0)
                       + t0)).astype(jnp.float32)
    G = jax.lax.dot_general(ST, U, (((0,), (0,)), ((), ())),
                            preferred_element_type=jnp.float32,
                            precision=HIGHEST)
    ir = jax.lax.broadcasted_iota(jnp.int32, (CAP, CAP), 0)
    ic = jax.lax.broadcasted_iota(jnp.int32, (CAP, CAP), 1)
    keep = jnp.logical_and(ir == ic, lwi_last < t0).astype(jnp.float32)
    slots_ref[...] = G + jax.lax.dot_general(
        keep, slots_ref[...], (((1,), (0,)), ((), ())),
        preferred_element_type=jnp.float32, precision=HIGHEST)


def _stage_c(IMP, LW, U, interpret=False):
    return pl.pallas_call(
        _combine_kernel,
        grid=(NB,),
        in_specs=[
            pl.BlockSpec((BM, CAP), lambda b: (b, 0)),
            pl.BlockSpec((BM, CAP), lambda b: (b, 0)),
            pl.BlockSpec((BM, D_MODEL), lambda b: (b, 0)),
        ],
        out_specs=pl.BlockSpec((BM, D_MODEL), lambda b: (b, 0)),
        out_shape=jax.ShapeDtypeStruct((BATCH, D_MODEL), jnp.float32),
        scratch_shapes=[
            pltpu.VMEM((CAP, D_MODEL), jnp.float32),
        ],
        interpret=interpret,
    )(IMP, LW, U)


def kernel(x, W_up, b_up, W_imp, b_imp):
    P = _stage_p(x, W_imp, b_imp)
    IMP, LW = _sc_recurrence(P.reshape(BATCH))
    U = _stage_u(x, W_up, b_up)
    return _stage_c(IMP, LW, U)


# speedup vs baseline: 5.6804x; 1.1494x over previous
import functools

import jax
import jax.numpy as jnp
from jax import lax
from jax.experimental import pallas as pl
from jax.experimental.pallas import tpu as pltpu
from jax.experimental.pallas import tpu_sc as plsc

D_MODEL = 1024
CAP = 64
BATCH = 4096
BMA = 512
NBA = BATCH // BMA
BM = 256
NB = BATCH // BM
SB = 512
NSB = BATCH // SB
L = 16
NV = CAP // L

NEG = float(jnp.finfo(jnp.float32).min)
HIGHEST = jax.lax.Precision.HIGHEST


def _imp_kernel(x_ref, wimp_ref, bimp_ref, p_ref):
    z = jax.lax.dot_general(wimp_ref[...], x_ref[...], (((1,), (1,)), ((), ())),
                            preferred_element_type=jnp.float32,
                            precision=HIGHEST)
    p = jax.nn.sigmoid(z + bimp_ref[0, 0])
    p_ref[...] = p.reshape(1, 1, BMA)


def _stage_p(x, W_imp, b_imp, interpret=False):
    return pl.pallas_call(
        _imp_kernel,
        grid=(NBA,),
        in_specs=[
            pl.BlockSpec((BMA, D_MODEL), lambda b: (b, 0)),
            pl.BlockSpec((1, D_MODEL), lambda b: (0, 0)),
            pl.BlockSpec(memory_space=pltpu.SMEM),
        ],
        out_specs=pl.BlockSpec((1, 1, BMA), lambda b: (b, 0, 0)),
        out_shape=jax.ShapeDtypeStruct((NBA, 1, BMA), jnp.float32),
        interpret=interpret,
    )(x, W_imp, b_imp.reshape(1, 1))


def _up_kernel(x_ref, wup_ref, bup_ref, u_ref):
    u = jax.lax.dot_general(x_ref[...], wup_ref[...], (((1,), (1,)), ((), ())),
                            preferred_element_type=jnp.float32,
                            precision=HIGHEST)
    u_ref[...] = u + bup_ref[...]


def _stage_u(x, W_up, b_up, interpret=False):
    return pl.pallas_call(
        _up_kernel,
        grid=(NBA,),
        in_specs=[
            pl.BlockSpec((BMA, D_MODEL), lambda b: (b, 0)),
            pl.BlockSpec((D_MODEL, D_MODEL), lambda b: (0, 0)),
            pl.BlockSpec((1, D_MODEL), lambda b: (0, 0)),
        ],
        out_specs=pl.BlockSpec((BMA, D_MODEL), lambda b: (b, 0)),
        out_shape=jax.ShapeDtypeStruct((BATCH, D_MODEL), jnp.float32),
        interpret=interpret,
    )(x, W_up, b_up.reshape(1, D_MODEL))


def _sc_recurrence(P):
    mesh = plsc.VectorSubcoreMesh(core_axis_name="c", subcore_axis_name="s")

    @functools.partial(
        pl.kernel,
        mesh=mesh,
        out_type=[
            jax.ShapeDtypeStruct((BATCH * CAP,), jnp.float32),
            jax.ShapeDtypeStruct((BATCH * CAP,), jnp.float32),
        ],
        scratch_types=[
            pltpu.VMEM((BATCH,), jnp.float32),
            pltpu.VMEM((SB * CAP,), jnp.float32),
            pltpu.VMEM((SB * CAP,), jnp.float32),
        ],
    )
    def run(p_hbm, imp_hbm, lw_hbm, p_v, impbuf, lwbuf):
        cid = lax.axis_index("c")
        sid = lax.axis_index("s")

        def gath(x, idx):
            return lax.gather(
                x, idx.reshape(L, 1),
                lax.GatherDimensionNumbers(
                    offset_dims=(), collapsed_slice_dims=(0,),
                    start_index_map=(0,)),
                slice_sizes=(1,),
                mode=lax.GatherScatterMode.PROMISE_IN_BOUNDS)

        def lane_min_splat(x):
            iota = lax.broadcasted_iota(jnp.int32, (L,), 0)
            for k in range(4):
                x = jnp.minimum(x, gath(x, iota ^ (1 << k)))
            return x

        @pl.when(jnp.logical_and(cid == 0, sid == 0))
        def _():
            pltpu.sync_copy(p_hbm, p_v)
            lanes = [
                lax.broadcasted_iota(jnp.int32, (L,), 0) + L * k
                for k in range(NV)
            ]
            zero = jnp.zeros((L,), jnp.float32)

            def step_in_chunk(bk, i, carry):
                imp = carry[0:NV]
                ages = carry[NV:2 * NV]
                lw = carry[2 * NV:3 * NV]
                t = bk * SB + i
                t_f = t.astype(jnp.float32)
                cb = (t // L) * L
                chunk = p_v[pl.ds(cb, L)]
                p_s = lax.gather(
                    chunk,
                    jnp.full((L, 1), t - cb, jnp.int32),
                    lax.GatherDimensionNumbers(
                        offset_dims=(),
                        collapsed_slice_dims=(0,),
                        start_index_map=(0,)),
                    slice_sizes=(1,),
                    mode=lax.GatherScatterMode.PROMISE_IN_BOUNDS)
                m = jnp.minimum(jnp.minimum(imp[0], imp[1]),
                                jnp.minimum(imp[2], imp[3]))
                m_s = lane_min_splat(m)
                cand = [jnp.where(imp[k] == m_s, lanes[k], CAP)
                        for k in range(NV)]
                c = jnp.minimum(jnp.minimum(cand[0], cand[1]),
                                jnp.minimum(cand[2], cand[3]))
                idx_s = lane_min_splat(c)
                one_i = jnp.ones((L,), jnp.int32)
                zero_i = jnp.zeros((L,), jnp.int32)
                should_i = jnp.maximum(
                    jnp.where(p_s > 0.1, one_i, zero_i),
                    jnp.where(m_s <= 0.0, one_i, zero_i))
                newimp = jnp.maximum(p_s, 0.1)
                out = []
                base = i * CAP
                for k in range(NV):
                    sel_i = jnp.where(lanes[k] == idx_s,
                                      one_i, zero_i) * should_i
                    ik = jnp.where(sel_i == 1, newimp, imp[k])
                    ak = jnp.where(sel_i >= 1, zero, ages[k]) + 1.0
                    ik = jnp.where(ak > 50.0, ik * 0.9, ik)
                    wk = jnp.where(sel_i > 0, jnp.full((L,), t_f), lw[k])
                    impbuf[pl.ds(base + L * k, L)] = ik
                    lwbuf[pl.ds(base + L * k, L)] = wk
                    out.append((ik, ak, wk))
                return tuple([o[0] for o in out] + [o[1] for o in out]
                             + [o[2] for o in out])

            carry = tuple([jnp.zeros((L,), jnp.float32)] * (2 * NV)
                          + [jnp.full((L,), -1.0)] * NV)
            for bk in range(NSB):
                carry = lax.fori_loop(
                    0, SB, functools.partial(step_in_chunk, bk), carry)
                pltpu.sync_copy(
                    impbuf, imp_hbm.at[pl.ds(bk * SB * CAP, SB * CAP)])
                pltpu.sync_copy(
                    lwbuf, lw_hbm.at[pl.ds(bk * SB * CAP, SB * CAP)])

    impf, lwf = run(P)
    return impf.reshape(BATCH, CAP), lwf.reshape(BATCH, CAP)


def _combine_kernel(imp_ref, lw_ref, u_ref, out_ref, slots_ref):
    b = pl.program_id(0)
    t0 = b * BM

    @pl.when(b == 0)
    def _init():
        slots_ref[...] = jnp.zeros((CAP, D_MODEL), jnp.float32)

    IMP = imp_ref[...]
    LW = lw_ref[...]
    occ = LW >= 0.0
    logits = jnp.where(occ, IMP, NEG)
    mx = jnp.max(logits, axis=1, keepdims=True)
    e = jnp.exp(logits - mx)
    w = e / jnp.sum(e, axis=1, keepdims=True)

    LWi = LW.astype(jnp.int32)
    in_blk = LWi >= t0
    M1 = jnp.where(in_blk, 0.0, w)
    srow = jax.lax.broadcasted_iota(jnp.int32, (BM, CAP), 0) + t0
    H_T = (LWi == srow).astype(jnp.float32)
    WG = jax.lax.dot_general(w, H_T, (((1,), (1,)), ((), ())),
                             preferred_element_type=jnp.float32,
                             precision=HIGHEST)
    LG = jax.lax.dot_general(LW, H_T, (((1,), (1,)), ((), ())),
                             preferred_element_type=jnp.float32,
                             precision=HIGHEST)
    krow = (jax.lax.broadcasted_iota(jnp.int32, (BM, BM), 1)
            + t0).astype(jnp.float32)
    M2 = jnp.where(jnp.abs(LG - krow) < 0.5, WG, 0.0)
    U = u_ref[...]
    out = jax.lax.dot_general(M1, slots_ref[...], (((1,), (0,)), ((), ())),
                              preferred_element_type=jnp.float32)
    out = out + jax.lax.dot_general(M2, U, (((1,), (0,)), ((), ())),
                                    preferred_element_type=jnp.float32)
    out_ref[...] = out

    lwi_last = LWi[BM - 1:BM, :]
    ST = (lwi_last == (jax.lax.broadcasted_iota(jnp.int32, (BM, CAP), 0)
                       + t0)).astype(jnp.float32)
    G = jax.lax.dot_general(ST, U, (((0,), (0,)), ((), ())),
                            preferred_element_type=jnp.float32)
    ir = jax.lax.broadcasted_iota(jnp.int32, (CAP, CAP), 0)
    ic = jax.lax.broadcasted_iota(jnp.int32, (CAP, CAP), 1)
    keep = jnp.logical_and(ir == ic, lwi_last < t0).astype(jnp.float32)
    slots_ref[...] = G + jax.lax.dot_general(
        keep, slots_ref[...], (((1,), (0,)), ((), ())),
        preferred_element_type=jnp.float32)


def _stage_c(IMP, LW, U, interpret=False):
    return pl.pallas_call(
        _combine_kernel,
        grid=(NB,),
        in_specs=[
            pl.BlockSpec((BM, CAP), lambda b: (b, 0)),
            pl.BlockSpec((BM, CAP), lambda b: (b, 0)),
            pl.BlockSpec((BM, D_MODEL), lambda b: (b, 0)),
        ],
        out_specs=pl.BlockSpec((BM, D_MODEL), lambda b: (b, 0)),
        out_shape=jax.ShapeDtypeStruct((BATCH, D_MODEL), jnp.float32),
        scratch_shapes=[
            pltpu.VMEM((CAP, D_MODEL), jnp.float32),
        ],
        interpret=interpret,
    )(IMP, LW, U)


def kernel(x, W_up, b_up, W_imp, b_imp):
    P = _stage_p(x, W_imp, b_imp)
    IMP, LW = _sc_recurrence(P.reshape(BATCH))
    U = _stage_u(x, W_up, b_up)
    return _stage_c(IMP, LW, U)
